# Initial kernel scaffold; baseline (speedup 1.0000x reference)
import jax
import jax.numpy as jnp
from jax.experimental import pallas as pl


def kernel(logits):
    raise NotImplementedError("write your pallas kernel here")


# speedup vs baseline: 1034.9439x; 1034.9439x over previous
import numpy as np
import jax
import jax.numpy as jnp
from jax import lax
from jax.experimental import pallas as pl
from jax.experimental.pallas import tpu as pltpu
from jax.experimental.pallas import tpu_sc as plsc

B = 32
V = 1_000_000
C = 512
NC = (V + C - 1) // C
NCP = 2048
W = 65536
NBLK = NCP // (W // C)
K = 50
NSEL = 56
NCAND = 64
TEMP = np.float32(0.9)
TOP_P = np.float32(0.9)
NEG_INF = np.float32(-np.inf)
IMAX = np.int32(2**31 - 1)
NVC = C // 16


V_CLAMP = 999552


def _cmax_body(x_ref, o_ref):
    x = x_ref[:, 0, :]
    cols = [jnp.max(x[:, j * C:(j + 1) * C], axis=1, keepdims=True)
            for j in range(W // C)]
    o_ref[...] = jnp.concatenate(cols, axis=1)

    JS = (NC - 1) % (W // C)

    @pl.when(pl.program_id(0) == NBLK - 1)
    def _():
        lane = lax.broadcasted_iota(jnp.int32, (B, C), 1)
        xl = jnp.where(lane < V - (NC - 1) * C, x[:, JS * C:(JS + 1) * C],
                       NEG_INF)
        o_ref[:, JS:JS + 1] = jnp.max(xl, axis=1, keepdims=True)


def _select_body(m_ref, startg_ref, baseg_ref, lowg_ref, thr_ref):
    lane = lax.broadcasted_iota(jnp.int32, (B, NCP), 1)
    M = jnp.where(lane < NC, m_ref[...], NEG_INF)
    lane64 = lax.broadcasted_iota(jnp.int32, (B, 64), 1)
    grp1024 = lax.broadcasted_iota(jnp.int32, (B, 1024), 1) // 16

    def step(i, carry):
        Mw, ids, low, base, thr = carry
        m = jnp.max(Mw, axis=1, keepdims=True)
        idx = jnp.min(jnp.where(Mw == m, lane, jnp.int32(NCP)), axis=1,
                      keepdims=True)
        ids = jnp.where(lane64 == i, idx, ids)
        low = jnp.where(grp1024 == i, idx * C, low)
        base = jnp.where(grp1024 == i, jnp.minimum(idx * C, V_CLAMP), base)
        Mw = jnp.where(lane == idx, NEG_INF, Mw)
        thr = jnp.broadcast_to(m, (B, 16))
        return Mw, ids, low, base, thr

    _, ids, low, base, thr = lax.fori_loop(
        0, NSEL, step,
        (M, jnp.zeros((B, 64), jnp.int32), jnp.zeros((B, 1024), jnp.int32),
         jnp.zeros((B, 1024), jnp.int32), jnp.zeros((B, 16), jnp.float32)))
    pad_chunk = (lane64 - NSEL) * 37
    chunk = jnp.where(lane64 < NSEL, ids, pad_chunk)
    startg_ref[...] = jnp.minimum(chunk * C, V_CLAMP)
    baseg_ref[...] = base
    lowg_ref[...] = low
    u = lax.bitcast_convert_type(thr, jnp.uint32)
    mono = jnp.where(u >= jnp.uint32(0x80000000),
                     jnp.uint32(0xFFFFFFFF) - u, u + jnp.uint32(0x80000000))
    mono = mono - jnp.uint32(8)
    u2 = jnp.where(mono >= jnp.uint32(0x80000000),
                   mono - jnp.uint32(0x80000000), jnp.uint32(0xFFFFFFFF) - mono)
    thr_ref[...] = lax.bitcast_convert_type(u2, jnp.float32)


def _sc_body(data_hbm, startg_hbm, out_hbm, start_v, rows_v, sem):
    wid = lax.axis_index("s") * 2 + lax.axis_index("c")
    pltpu.sync_copy(startg_hbm.at[wid], start_v)
    lane = jnp.arange(16, dtype=jnp.int32)

    for grp in range((NSEL + 15) // 16):
        vr = start_v[pl.ds(grp * 16, 16)]
        for l in range(16):
            j = grp * 16 + l
            if j >= NSEL:
                break
            st = pl.multiple_of(vr[l], 128)
            pltpu.async_copy(data_hbm.at[wid, 0, pl.ds(st, C)],
                             rows_v.at[pl.ds(j * C, C)], sem)
    pltpu.make_async_copy(
        data_hbm.at[0, 0, pl.ds(0, NSEL * C)], rows_v, sem).wait()
    pltpu.sync_copy(rows_v, out_hbm.at[wid, 0])


def _threefry_bits(c2):
    ks0 = jnp.uint32(0)
    ks1 = jnp.uint32(42)
    ks2 = jnp.uint32(0 ^ 42 ^ 0x1BD11BDA)
    ks = [ks0, ks1, ks2]
    rot = [[13, 15, 26, 6], [17, 29, 16, 24]]
    x0 = jnp.zeros_like(c2) + ks0
    x1 = c2 + ks1
    for i in range(5):
        for r in rot[i % 2]:
            x0 = x0 + x1
            x1 = (x1 << r) | (x1 >> (32 - r))
            x1 = x0 ^ x1
        x0 = x0 + ks[(i + 1) % 3]
        x1 = x1 + ks[(i + 2) % 3] + jnp.uint32(i + 1)
    return x0 ^ x1


def _final_body(gath_ref, baseg_ref, lowg_ref, thr_ref, o_ref, vw_ref, gg_ref):
    x = gath_ref[:, 0, :]
    thr = thr_ref[:, :1]
    off512 = lax.broadcasted_iota(jnp.int32, (B, C), 1)
    gcols = []
    lcols = []
    for r in range(NSEL):
        gcols.append(baseg_ref[:, r * 16:r * 16 + 1] + off512)
        lcols.append(jnp.broadcast_to(lowg_ref[:, r * 16:r * 16 + 1], (B, C)))
    gg = jnp.concatenate(gcols, axis=1)
    lo = jnp.concatenate(lcols, axis=1)
    ok = (x >= thr) & (gg >= lo) & (gg < V)
    vw_ref[...] = jnp.where(ok, x, NEG_INF) / TEMP
    gg_ref[...] = jnp.where(ok, gg, IMAX)

    lane = lax.broadcasted_iota(jnp.int32, (B, 64), 1)

    def wstep(i, carry):
        sv, sg = carry
        vw = vw_ref[...]
        ggm = gg_ref[...]
        m = jnp.max(vw, axis=1, keepdims=True)
        gsel = jnp.min(jnp.where(vw == m, ggm, IMAX), axis=1, keepdims=True)
        sv = jnp.where(lane == i, m, sv)
        sg = jnp.where(lane == i, gsel, sg)
        vw_ref[...] = jnp.where((vw == m) & (ggm == gsel), NEG_INF, vw)
        return sv, sg

    sv0 = jnp.minimum(x[:, :64], NEG_INF)
    sg0 = jnp.maximum(gg[:, :64], IMAX)
    vraw, g = lax.fori_loop(0, K, wstep, (sv0, sg0))
    v = vraw

    def stepA(i, carry):
        vw, kept = carry
        m = jnp.max(vw, axis=1, keepdims=True)
        gsel = jnp.min(jnp.where(vw == m, g, IMAX), axis=1, keepdims=True)
        hit = (vw == m) & (g == gsel)
        kept = jnp.where(hit, jnp.int32(1), kept)
        vw = jnp.where(hit, NEG_INF, vw)
        return vw, kept

    _, kept_i = lax.fori_loop(0, K, stepA, (v, g * 0))
    kept = kept_i == 1

    vk = jnp.where(kept, v, NEG_INF)

    def stepB(i, carry):
        vw, sv, sg = carry
        m = jnp.max(vw, axis=1, keepdims=True)
        gsel = jnp.max(jnp.where(vw == m, g, jnp.int32(-1)), axis=1,
                       keepdims=True)
        sv = jnp.where(lane == i, m, sv)
        sg = jnp.where(lane == i, gsel, sg)
        vw = jnp.where((vw == m) & (g == gsel), NEG_INF, vw)
        return vw, sv, sg

    _, sv, sg = lax.fori_loop(
        0, K, stepB, (vk, jnp.minimum(v, NEG_INF), jnp.maximum(g, IMAX)))
    valid = lane < K

    m0 = jnp.max(sv, axis=1, keepdims=True)
    e = jnp.where(valid, jnp.exp(sv - m0), np.float32(0.0))
    s = jnp.sum(e, axis=1, keepdims=True)
    p = e / s

    def cstep(j, carry):
        run, cprev = carry
        cprev = jnp.where(lane == j, run, cprev)
        pj = jnp.sum(jnp.where(lane == j, p, np.float32(0.0)), axis=1,
                     keepdims=True)
        return run + pj, cprev

    _, cprev = lax.fori_loop(
        0, K, cstep,
        (jnp.sum(p, axis=1, keepdims=True) * np.float32(0.0),
         p * np.float32(0.0)))
    remove = (lane > 0) & (cprev > TOP_P)
    keep = valid & jnp.logical_not(remove)

    row = lax.broadcasted_iota(jnp.int32, (B, 64), 0)
    pflat = (row * V + jnp.where(keep, sg, 0)).astype(jnp.uint32)
    bits = _threefry_bits(pflat)
    fb = (bits >> 9) | jnp.uint32(np.float32(1.0).view(np.uint32))
    f = lax.bitcast_convert_type(fb, jnp.float32) - np.float32(1.0)
    tiny = np.float32(np.finfo(np.float32).tiny)
    u = jnp.maximum(tiny, f * (np.float32(1.0) - tiny) + tiny)
    gum = -jnp.log(-jnp.log(u))
    score = jnp.where(keep, sv + gum, NEG_INF)
    ms = jnp.max(score, axis=1, keepdims=True)
    wg = jnp.min(jnp.where(score == ms, sg, IMAX), axis=1, keepdims=True)
    o_ref[...] = wg


def kernel(logits):
    cmax = pl.pallas_call(
        _cmax_body,
        grid=(NBLK,),
        in_specs=[pl.BlockSpec((B, 1, W), lambda i: (0, 0, i))],
        out_specs=pl.BlockSpec((B, W // C), lambda i: (0, i)),
        out_shape=jax.ShapeDtypeStruct((B, NCP), jnp.float32),
    )(logits)

    startg, baseg, lowg, thr = pl.pallas_call(
        _select_body,
        out_shape=[
            jax.ShapeDtypeStruct((B, 64), jnp.int32),
            jax.ShapeDtypeStruct((B, 1024), jnp.int32),
            jax.ShapeDtypeStruct((B, 1024), jnp.int32),
            jax.ShapeDtypeStruct((B, 16), jnp.float32),
        ],
    )(cmax)

    mesh = plsc.VectorSubcoreMesh(core_axis_name="c", subcore_axis_name="s")
    gathered = pl.kernel(
        _sc_body,
        mesh=mesh,
        out_type=jax.ShapeDtypeStruct((B, 1, NSEL * C), jnp.float32),
        scratch_types=[
            pltpu.VMEM((64,), jnp.int32),
            pltpu.VMEM((NSEL * C,), jnp.float32),
            pltpu.SemaphoreType.DMA,
        ],
    )(logits, startg)

    tok = pl.pallas_call(
        _final_body,
        out_shape=jax.ShapeDtypeStruct((B, 1), jnp.int32),
        scratch_shapes=[
            pltpu.VMEM((B, NSEL * C), jnp.float32),
            pltpu.VMEM((B, NSEL * C), jnp.int32),
        ],
    )(gathered, baseg, lowg, thr)
    return tok


# speedup vs baseline: 1258.8817x; 1.2164x over previous
import numpy as np
import jax
import jax.numpy as jnp
from jax import lax
from jax.experimental import pallas as pl
from jax.experimental.pallas import tpu as pltpu
from jax.experimental.pallas import tpu_sc as plsc

B = 32
V = 1_000_000
C = 256
NC = (V + C - 1) // C
NCP = 4096
W = 65536
NBLK = NCP // (W // C)
K = 50
NSEL = 56
NCAND = 64
TEMP = np.float32(0.9)
TOP_P = np.float32(0.9)
NEG_INF = np.float32(-np.inf)
IMAX = np.int32(2**31 - 1)
NVC = C // 16


V_CLAMP = 999808


def _cmax_body(x_ref, o_ref):
    x = x_ref[:, 0, :]
    cols = [jnp.max(x[:, j * C:(j + 1) * C], axis=1, keepdims=True)
            for j in range(W // C)]
    o_ref[...] = jnp.concatenate(cols, axis=1)

    JS = (NC - 1) % (W // C)

    @pl.when(pl.program_id(0) == NBLK - 1)
    def _():
        lane = lax.broadcasted_iota(jnp.int32, (B, C), 1)
        xl = jnp.where(lane < V - (NC - 1) * C, x[:, JS * C:(JS + 1) * C],
                       NEG_INF)
        o_ref[:, JS:JS + 1] = jnp.max(xl, axis=1, keepdims=True)


def _select_body(m_ref, startg_ref, baseg_ref, lowg_ref, thr_ref):
    lane = lax.broadcasted_iota(jnp.int32, (B, NCP), 1)
    M = jnp.where(lane < NC, m_ref[...], NEG_INF)
    lane64 = lax.broadcasted_iota(jnp.int32, (B, 64), 1)
    grp1024 = lax.broadcasted_iota(jnp.int32, (B, 1024), 1) // 16

    def step(i, carry):
        Mw, ids, low, base, thr = carry
        m = jnp.max(Mw, axis=1, keepdims=True)
        idx = jnp.min(jnp.where(Mw == m, lane, jnp.int32(NCP)), axis=1,
                      keepdims=True)
        ids = jnp.where(lane64 == i, idx, ids)
        low = jnp.where(grp1024 == i, idx * C, low)
        base = jnp.where(grp1024 == i, jnp.minimum(idx * C, V_CLAMP), base)
        Mw = jnp.where(lane == idx, NEG_INF, Mw)
        thr = jnp.broadcast_to(m, (B, 16))
        return Mw, ids, low, base, thr

    _, ids, low, base, thr = lax.fori_loop(
        0, NSEL, step,
        (M, jnp.zeros((B, 64), jnp.int32), jnp.zeros((B, 1024), jnp.int32),
         jnp.zeros((B, 1024), jnp.int32), jnp.zeros((B, 16), jnp.float32)))
    pad_chunk = (lane64 - NSEL) * 37
    chunk = jnp.where(lane64 < NSEL, ids, pad_chunk)
    startg_ref[...] = jnp.minimum(chunk * C, V_CLAMP)
    baseg_ref[...] = base
    lowg_ref[...] = low
    u = lax.bitcast_convert_type(thr, jnp.uint32)
    mono = jnp.where(u >= jnp.uint32(0x80000000),
                     jnp.uint32(0xFFFFFFFF) - u, u + jnp.uint32(0x80000000))
    mono = mono - jnp.uint32(8)
    u2 = jnp.where(mono >= jnp.uint32(0x80000000),
                   mono - jnp.uint32(0x80000000), jnp.uint32(0xFFFFFFFF) - mono)
    thr_ref[...] = lax.bitcast_convert_type(u2, jnp.float32)


def _sc_body(data_hbm, startg_hbm, out_hbm, start_v, rows_v, sem):
    wid = lax.axis_index("s") * 2 + lax.axis_index("c")
    pltpu.sync_copy(startg_hbm.at[wid], start_v)
    lane = jnp.arange(16, dtype=jnp.int32)

    for grp in range((NSEL + 15) // 16):
        vr = start_v[pl.ds(grp * 16, 16)]
        for l in range(16):
            j = grp * 16 + l
            if j >= NSEL:
                break
            st = pl.multiple_of(vr[l], 128)
            pltpu.async_copy(data_hbm.at[wid, 0, pl.ds(st, C)],
                             rows_v.at[pl.ds(j * C, C)], sem)
    pltpu.make_async_copy(
        data_hbm.at[0, 0, pl.ds(0, NSEL * C)], rows_v, sem).wait()
    pltpu.sync_copy(rows_v, out_hbm.at[wid, 0])


def _threefry_bits(c2):
    ks0 = jnp.uint32(0)
    ks1 = jnp.uint32(42)
    ks2 = jnp.uint32(0 ^ 42 ^ 0x1BD11BDA)
    ks = [ks0, ks1, ks2]
    rot = [[13, 15, 26, 6], [17, 29, 16, 24]]
    x0 = jnp.zeros_like(c2) + ks0
    x1 = c2 + ks1
    for i in range(5):
        for r in rot[i % 2]:
            x0 = x0 + x1
            x1 = (x1 << r) | (x1 >> (32 - r))
            x1 = x0 ^ x1
        x0 = x0 + ks[(i + 1) % 3]
        x1 = x1 + ks[(i + 2) % 3] + jnp.uint32(i + 1)
    return x0 ^ x1


def _final_body(gath_ref, baseg_ref, lowg_ref, thr_ref, o_ref, vw_ref, gg_ref):
    x = gath_ref[:, 0, :]
    thr = thr_ref[:, :1]
    off512 = lax.broadcasted_iota(jnp.int32, (B, C), 1)
    gcols = []
    lcols = []
    for r in range(NSEL):
        gcols.append(baseg_ref[:, r * 16:r * 16 + 1] + off512)
        lcols.append(jnp.broadcast_to(lowg_ref[:, r * 16:r * 16 + 1], (B, C)))
    gg = jnp.concatenate(gcols, axis=1)
    lo = jnp.concatenate(lcols, axis=1)
    ok = (x >= thr) & (gg >= lo) & (gg < V)
    vw_ref[...] = jnp.where(ok, x, NEG_INF) / TEMP
    gg_ref[...] = jnp.where(ok, gg, IMAX)

    lane = lax.broadcasted_iota(jnp.int32, (B, 64), 1)

    def wstep(i, carry):
        sv, sg = carry
        vw = vw_ref[...]
        ggm = gg_ref[...]
        m = jnp.max(vw, axis=1, keepdims=True)
        gsel = jnp.min(jnp.where(vw == m, ggm, IMAX), axis=1, keepdims=True)
        sv = jnp.where(lane == i, m, sv)
        sg = jnp.where(lane == i, gsel, sg)
        vw_ref[...] = jnp.where((vw == m) & (ggm == gsel), NEG_INF, vw)
        return sv, sg

    sv0 = jnp.minimum(x[:, :64], NEG_INF)
    sg0 = jnp.maximum(gg[:, :64], IMAX)
    vraw, g = lax.fori_loop(0, K, wstep, (sv0, sg0))
    v = vraw

    def stepA(i, carry):
        vw, kept = carry
        m = jnp.max(vw, axis=1, keepdims=True)
        gsel = jnp.min(jnp.where(vw == m, g, IMAX), axis=1, keepdims=True)
        hit = (vw == m) & (g == gsel)
        kept = jnp.where(hit, jnp.int32(1), kept)
        vw = jnp.where(hit, NEG_INF, vw)
        return vw, kept

    _, kept_i = lax.fori_loop(0, K, stepA, (v, g * 0))
    kept = kept_i == 1

    vk = jnp.where(kept, v, NEG_INF)

    def stepB(i, carry):
        vw, sv, sg = carry
        m = jnp.max(vw, axis=1, keepdims=True)
        gsel = jnp.max(jnp.where(vw == m, g, jnp.int32(-1)), axis=1,
                       keepdims=True)
        sv = jnp.where(lane == i, m, sv)
        sg = jnp.where(lane == i, gsel, sg)
        vw = jnp.where((vw == m) & (g == gsel), NEG_INF, vw)
        return vw, sv, sg

    _, sv, sg = lax.fori_loop(
        0, K, stepB, (vk, jnp.minimum(v, NEG_INF), jnp.maximum(g, IMAX)))
    valid = lane < K

    m0 = jnp.max(sv, axis=1, keepdims=True)
    e = jnp.where(valid, jnp.exp(sv - m0), np.float32(0.0))
    s = jnp.sum(e, axis=1, keepdims=True)
    p = e / s

    def cstep(j, carry):
        run, cprev = carry
        cprev = jnp.where(lane == j, run, cprev)
        pj = jnp.sum(jnp.where(lane == j, p, np.float32(0.0)), axis=1,
                     keepdims=True)
        return run + pj, cprev

    _, cprev = lax.fori_loop(
        0, K, cstep,
        (jnp.sum(p, axis=1, keepdims=True) * np.float32(0.0),
         p * np.float32(0.0)))
    remove = (lane > 0) & (cprev > TOP_P)
    keep = valid & jnp.logical_not(remove)

    row = lax.broadcasted_iota(jnp.int32, (B, 64), 0)
    pflat = (row * V + jnp.where(keep, sg, 0)).astype(jnp.uint32)
    bits = _threefry_bits(pflat)
    fb = (bits >> 9) | jnp.uint32(np.float32(1.0).view(np.uint32))
    f = lax.bitcast_convert_type(fb, jnp.float32) - np.float32(1.0)
    tiny = np.float32(np.finfo(np.float32).tiny)
    u = jnp.maximum(tiny, f * (np.float32(1.0) - tiny) + tiny)
    gum = -jnp.log(-jnp.log(u))
    score = jnp.where(keep, sv + gum, NEG_INF)
    ms = jnp.max(score, axis=1, keepdims=True)
    wg = jnp.min(jnp.where(score == ms, sg, IMAX), axis=1, keepdims=True)
    o_ref[...] = wg


def kernel(logits):
    cmax = pl.pallas_call(
        _cmax_body,
        grid=(NBLK,),
        in_specs=[pl.BlockSpec((B, 1, W), lambda i: (0, 0, i))],
        out_specs=pl.BlockSpec((B, W // C), lambda i: (0, i)),
        out_shape=jax.ShapeDtypeStruct((B, NCP), jnp.float32),
    )(logits)

    startg, baseg, lowg, thr = pl.pallas_call(
        _select_body,
        out_shape=[
            jax.ShapeDtypeStruct((B, 64), jnp.int32),
            jax.ShapeDtypeStruct((B, 1024), jnp.int32),
            jax.ShapeDtypeStruct((B, 1024), jnp.int32),
            jax.ShapeDtypeStruct((B, 16), jnp.float32),
        ],
    )(cmax)

    mesh = plsc.VectorSubcoreMesh(core_axis_name="c", subcore_axis_name="s")
    gathered = pl.kernel(
        _sc_body,
        mesh=mesh,
        out_type=jax.ShapeDtypeStruct((B, 1, NSEL * C), jnp.float32),
        scratch_types=[
            pltpu.VMEM((64,), jnp.int32),
            pltpu.VMEM((NSEL * C,), jnp.float32),
            pltpu.SemaphoreType.DMA,
        ],
    )(logits, startg)

    tok = pl.pallas_call(
        _final_body,
        out_shape=jax.ShapeDtypeStruct((B, 1), jnp.int32),
        scratch_shapes=[
            pltpu.VMEM((B, NSEL * C), jnp.float32),
            pltpu.VMEM((B, NSEL * C), jnp.int32),
        ],
    )(gathered, baseg, lowg, thr)
    return tok


# speedup vs baseline: 1278.6931x; 1.0157x over previous
import numpy as np
import jax
import jax.numpy as jnp
from jax import lax
from jax.experimental import pallas as pl
from jax.experimental.pallas import tpu as pltpu
from jax.experimental.pallas import tpu_sc as plsc

B = 32
V = 1_000_000
C = 256
NC = (V + C - 1) // C
NCP = 4096
W = 131072
NBLK = NCP // (W // C)
K = 50
NSEL = 56
NCAND = 64
TEMP = np.float32(0.9)
TOP_P = np.float32(0.9)
NEG_INF = np.float32(-np.inf)
IMAX = np.int32(2**31 - 1)
NVC = C // 16


V_CLAMP = 999808


def _cmax_body(x_ref, o_ref):
    x = x_ref[:, 0, :]
    cols = [jnp.max(x[:, j * C:(j + 1) * C], axis=1, keepdims=True)
            for j in range(W // C)]
    o_ref[...] = jnp.concatenate(cols, axis=1)

    JS = (NC - 1) % (W // C)

    @pl.when(pl.program_id(0) == NBLK - 1)
    def _():
        lane = lax.broadcasted_iota(jnp.int32, (B, C), 1)
        xl = jnp.where(lane < V - (NC - 1) * C, x[:, JS * C:(JS + 1) * C],
                       NEG_INF)
        o_ref[:, JS:JS + 1] = jnp.max(xl, axis=1, keepdims=True)


def _select_body(m_ref, startg_ref, baseg_ref, lowg_ref, thr_ref):
    lane = lax.broadcasted_iota(jnp.int32, (B, NCP), 1)
    M = jnp.where(lane < NC, m_ref[...], NEG_INF)
    lane64 = lax.broadcasted_iota(jnp.int32, (B, 64), 1)
    grp1024 = lax.broadcasted_iota(jnp.int32, (B, 1024), 1) // 16

    def step(i, carry):
        Mw, ids, low, base, thr = carry
        m = jnp.max(Mw, axis=1, keepdims=True)
        idx = jnp.min(jnp.where(Mw == m, lane, jnp.int32(NCP)), axis=1,
                      keepdims=True)
        ids = jnp.where(lane64 == i, idx, ids)
        low = jnp.where(grp1024 == i, idx * C, low)
        base = jnp.where(grp1024 == i, jnp.minimum(idx * C, V_CLAMP), base)
        Mw = jnp.where(lane == idx, NEG_INF, Mw)
        thr = jnp.broadcast_to(m, (B, 16))
        return Mw, ids, low, base, thr

    _, ids, low, base, thr = lax.fori_loop(
        0, NSEL, step,
        (M, jnp.zeros((B, 64), jnp.int32), jnp.zeros((B, 1024), jnp.int32),
         jnp.zeros((B, 1024), jnp.int32), jnp.zeros((B, 16), jnp.float32)))
    pad_chunk = (lane64 - NSEL) * 37
    chunk = jnp.where(lane64 < NSEL, ids, pad_chunk)
    startg_ref[...] = jnp.minimum(chunk * C, V_CLAMP)
    baseg_ref[...] = base
    lowg_ref[...] = low
    u = lax.bitcast_convert_type(thr, jnp.uint32)
    mono = jnp.where(u >= jnp.uint32(0x80000000),
                     jnp.uint32(0xFFFFFFFF) - u, u + jnp.uint32(0x80000000))
    mono = mono - jnp.uint32(8)
    u2 = jnp.where(mono >= jnp.uint32(0x80000000),
                   mono - jnp.uint32(0x80000000), jnp.uint32(0xFFFFFFFF) - mono)
    thr_ref[...] = lax.bitcast_convert_type(u2, jnp.float32)


def _sc_body(data_hbm, startg_hbm, out_hbm, start_v, rows_v, sem):
    wid = lax.axis_index("s") * 2 + lax.axis_index("c")
    pltpu.sync_copy(startg_hbm.at[wid], start_v)
    lane = jnp.arange(16, dtype=jnp.int32)

    for grp in range((NSEL + 15) // 16):
        vr = start_v[pl.ds(grp * 16, 16)]
        for l in range(16):
            j = grp * 16 + l
            if j >= NSEL:
                break
            st = pl.multiple_of(vr[l], 128)
            pltpu.async_copy(data_hbm.at[wid, 0, pl.ds(st, C)],
                             rows_v.at[pl.ds(j * C, C)], sem)
    pltpu.make_async_copy(
        data_hbm.at[0, 0, pl.ds(0, NSEL * C)], rows_v, sem).wait()
    pltpu.sync_copy(rows_v, out_hbm.at[wid, 0])


def _threefry_bits(c2):
    ks0 = jnp.uint32(0)
    ks1 = jnp.uint32(42)
    ks2 = jnp.uint32(0 ^ 42 ^ 0x1BD11BDA)
    ks = [ks0, ks1, ks2]
    rot = [[13, 15, 26, 6], [17, 29, 16, 24]]
    x0 = jnp.zeros_like(c2) + ks0
    x1 = c2 + ks1
    for i in range(5):
        for r in rot[i % 2]:
            x0 = x0 + x1
            x1 = (x1 << r) | (x1 >> (32 - r))
            x1 = x0 ^ x1
        x0 = x0 + ks[(i + 1) % 3]
        x1 = x1 + ks[(i + 2) % 3] + jnp.uint32(i + 1)
    return x0 ^ x1


def _final_body(gath_ref, baseg_ref, lowg_ref, thr_ref, o_ref, vw_ref, gg_ref):
    x = gath_ref[:, 0, :]
    thr = thr_ref[:, :1]
    off512 = lax.broadcasted_iota(jnp.int32, (B, C), 1)
    gcols = []
    lcols = []
    for r in range(NSEL):
        gcols.append(baseg_ref[:, r * 16:r * 16 + 1] + off512)
        lcols.append(jnp.broadcast_to(lowg_ref[:, r * 16:r * 16 + 1], (B, C)))
    gg = jnp.concatenate(gcols, axis=1)
    lo = jnp.concatenate(lcols, axis=1)
    ok = (x >= thr) & (gg >= lo) & (gg < V)
    vw_ref[...] = jnp.where(ok, x, NEG_INF) / TEMP
    gg_ref[...] = jnp.where(ok, gg, IMAX)

    lane = lax.broadcasted_iota(jnp.int32, (B, 64), 1)

    def wstep(i, carry):
        sv, sg = carry
        vw = vw_ref[...]
        ggm = gg_ref[...]
        m = jnp.max(vw, axis=1, keepdims=True)
        gsel = jnp.min(jnp.where(vw == m, ggm, IMAX), axis=1, keepdims=True)
        sv = jnp.where(lane == i, m, sv)
        sg = jnp.where(lane == i, gsel, sg)
        vw_ref[...] = jnp.where((vw == m) & (ggm == gsel), NEG_INF, vw)
        return sv, sg

    sv0 = jnp.minimum(x[:, :64], NEG_INF)
    sg0 = jnp.maximum(gg[:, :64], IMAX)
    vraw, g = lax.fori_loop(0, K, wstep, (sv0, sg0))
    v = vraw

    def stepA(i, carry):
        vw, kept = carry
        m = jnp.max(vw, axis=1, keepdims=True)
        gsel = jnp.min(jnp.where(vw == m, g, IMAX), axis=1, keepdims=True)
        hit = (vw == m) & (g == gsel)
        kept = jnp.where(hit, jnp.int32(1), kept)
        vw = jnp.where(hit, NEG_INF, vw)
        return vw, kept

    _, kept_i = lax.fori_loop(0, K, stepA, (v, g * 0))
    kept = kept_i == 1

    vk = jnp.where(kept, v, NEG_INF)

    def stepB(i, carry):
        vw, sv, sg = carry
        m = jnp.max(vw, axis=1, keepdims=True)
        gsel = jnp.max(jnp.where(vw == m, g, jnp.int32(-1)), axis=1,
                       keepdims=True)
        sv = jnp.where(lane == i, m, sv)
        sg = jnp.where(lane == i, gsel, sg)
        vw = jnp.where((vw == m) & (g == gsel), NEG_INF, vw)
        return vw, sv, sg

    _, sv, sg = lax.fori_loop(
        0, K, stepB, (vk, jnp.minimum(v, NEG_INF), jnp.maximum(g, IMAX)))
    valid = lane < K

    m0 = jnp.max(sv, axis=1, keepdims=True)
    e = jnp.where(valid, jnp.exp(sv - m0), np.float32(0.0))
    s = jnp.sum(e, axis=1, keepdims=True)
    p = e / s

    def cstep(j, carry):
        run, cprev = carry
        cprev = jnp.where(lane == j, run, cprev)
        pj = jnp.sum(jnp.where(lane == j, p, np.float32(0.0)), axis=1,
                     keepdims=True)
        return run + pj, cprev

    _, cprev = lax.fori_loop(
        0, K, cstep,
        (jnp.sum(p, axis=1, keepdims=True) * np.float32(0.0),
         p * np.float32(0.0)))
    remove = (lane > 0) & (cprev > TOP_P)
    keep = valid & jnp.logical_not(remove)

    row = lax.broadcasted_iota(jnp.int32, (B, 64), 0)
    pflat = (row * V + jnp.where(keep, sg, 0)).astype(jnp.uint32)
    bits = _threefry_bits(pflat)
    fb = (bits >> 9) | jnp.uint32(np.float32(1.0).view(np.uint32))
    f = lax.bitcast_convert_type(fb, jnp.float32) - np.float32(1.0)
    tiny = np.float32(np.finfo(np.float32).tiny)
    u = jnp.maximum(tiny, f * (np.float32(1.0) - tiny) + tiny)
    gum = -jnp.log(-jnp.log(u))
    score = jnp.where(keep, sv + gum, NEG_INF)
    ms = jnp.max(score, axis=1, keepdims=True)
    wg = jnp.min(jnp.where(score == ms, sg, IMAX), axis=1, keepdims=True)
    o_ref[...] = wg


def kernel(logits):
    cmax = pl.pallas_call(
        _cmax_body,
        grid=(NBLK,),
        in_specs=[pl.BlockSpec((B, 1, W), lambda i: (0, 0, i))],
        out_specs=pl.BlockSpec((B, W // C), lambda i: (0, i)),
        out_shape=jax.ShapeDtypeStruct((B, NCP), jnp.float32),
    )(logits)

    startg, baseg, lowg, thr = pl.pallas_call(
        _select_body,
        out_shape=[
            jax.ShapeDtypeStruct((B, 64), jnp.int32),
            jax.ShapeDtypeStruct((B, 1024), jnp.int32),
            jax.ShapeDtypeStruct((B, 1024), jnp.int32),
            jax.ShapeDtypeStruct((B, 16), jnp.float32),
        ],
    )(cmax)

    mesh = plsc.VectorSubcoreMesh(core_axis_name="c", subcore_axis_name="s")
    gathered = pl.kernel(
        _sc_body,
        mesh=mesh,
        out_type=jax.ShapeDtypeStruct((B, 1, NSEL * C), jnp.float32),
        scratch_types=[
            pltpu.VMEM((64,), jnp.int32),
            pltpu.VMEM((NSEL * C,), jnp.float32),
            pltpu.SemaphoreType.DMA,
        ],
    )(logits, startg)

    tok = pl.pallas_call(
        _final_body,
        out_shape=jax.ShapeDtypeStruct((B, 1), jnp.int32),
        scratch_shapes=[
            pltpu.VMEM((B, NSEL * C), jnp.float32),
            pltpu.VMEM((B, NSEL * C), jnp.int32),
        ],
    )(gathered, baseg, lowg, thr)
    return tok


# speedup vs baseline: 1343.4854x; 1.0507x over previous
import numpy as np
import jax
import jax.numpy as jnp
from jax import lax
from jax.experimental import pallas as pl
from jax.experimental.pallas import tpu as pltpu
from jax.experimental.pallas import tpu_sc as plsc

B = 32
V = 1_000_000
C = 128
NC = (V + C - 1) // C
NCP = 8192
W = 131072
NBLK = NCP // (W // C)
K = 50
NSEL = 56
NCAND = 64
TEMP = np.float32(0.9)
TOP_P = np.float32(0.9)
NEG_INF = np.float32(-np.inf)
IMAX = np.int32(2**31 - 1)
NVC = C // 16


V_CLAMP = 999936


def _cmax_body(x_ref, o_ref):
    x = x_ref[:, 0, :]
    cols = [jnp.max(x[:, j * C:(j + 1) * C], axis=1, keepdims=True)
            for j in range(W // C)]
    o_ref[...] = jnp.concatenate(cols, axis=1)

    JS = (NC - 1) % (W // C)

    @pl.when(pl.program_id(0) == NBLK - 1)
    def _():
        lane = lax.broadcasted_iota(jnp.int32, (B, C), 1)
        xl = jnp.where(lane < V - (NC - 1) * C, x[:, JS * C:(JS + 1) * C],
                       NEG_INF)
        o_ref[:, JS:JS + 1] = jnp.max(xl, axis=1, keepdims=True)


def _select_body(m_ref, startg_ref, baseg_ref, lowg_ref, thr_ref):
    lane = lax.broadcasted_iota(jnp.int32, (B, NCP), 1)
    M = jnp.where(lane < NC, m_ref[...], NEG_INF)
    lane64 = lax.broadcasted_iota(jnp.int32, (B, 64), 1)
    grp1024 = lax.broadcasted_iota(jnp.int32, (B, 1024), 1) // 16

    def step(i, carry):
        Mw, ids, low, base, thr = carry
        m = jnp.max(Mw, axis=1, keepdims=True)
        idx = jnp.min(jnp.where(Mw == m, lane, jnp.int32(NCP)), axis=1,
                      keepdims=True)
        ids = jnp.where(lane64 == i, idx, ids)
        low = jnp.where(grp1024 == i, idx * C, low)
        base = jnp.where(grp1024 == i, jnp.minimum(idx * C, V_CLAMP), base)
        Mw = jnp.where(lane == idx, NEG_INF, Mw)
        thr = jnp.broadcast_to(m, (B, 16))
        return Mw, ids, low, base, thr

    _, ids, low, base, thr = lax.fori_loop(
        0, NSEL, step,
        (M, jnp.zeros((B, 64), jnp.int32), jnp.zeros((B, 1024), jnp.int32),
         jnp.zeros((B, 1024), jnp.int32), jnp.zeros((B, 16), jnp.float32)))
    pad_chunk = (lane64 - NSEL) * 37
    chunk = jnp.where(lane64 < NSEL, ids, pad_chunk)
    startg_ref[...] = jnp.minimum(chunk * C, V_CLAMP)
    baseg_ref[...] = base
    lowg_ref[...] = low
    u = lax.bitcast_convert_type(thr, jnp.uint32)
    mono = jnp.where(u >= jnp.uint32(0x80000000),
                     jnp.uint32(0xFFFFFFFF) - u, u + jnp.uint32(0x80000000))
    mono = mono - jnp.uint32(8)
    u2 = jnp.where(mono >= jnp.uint32(0x80000000),
                   mono - jnp.uint32(0x80000000), jnp.uint32(0xFFFFFFFF) - mono)
    thr_ref[...] = lax.bitcast_convert_type(u2, jnp.float32)


def _sc_body(data_hbm, startg_hbm, out_hbm, start_v, rows_v, sem):
    wid = lax.axis_index("s") * 2 + lax.axis_index("c")
    pltpu.sync_copy(startg_hbm.at[wid], start_v)
    lane = jnp.arange(16, dtype=jnp.int32)

    for grp in range((NSEL + 15) // 16):
        vr = start_v[pl.ds(grp * 16, 16)]
        for l in range(16):
            j = grp * 16 + l
            if j >= NSEL:
                break
            st = pl.multiple_of(vr[l], 128)
            pltpu.async_copy(data_hbm.at[wid, 0, pl.ds(st, C)],
                             rows_v.at[pl.ds(j * C, C)], sem)
    pltpu.make_async_copy(
        data_hbm.at[0, 0, pl.ds(0, NSEL * C)], rows_v, sem).wait()
    pltpu.sync_copy(rows_v, out_hbm.at[wid, 0])


def _threefry_bits(c2):
    ks0 = jnp.uint32(0)
    ks1 = jnp.uint32(42)
    ks2 = jnp.uint32(0 ^ 42 ^ 0x1BD11BDA)
    ks = [ks0, ks1, ks2]
    rot = [[13, 15, 26, 6], [17, 29, 16, 24]]
    x0 = jnp.zeros_like(c2) + ks0
    x1 = c2 + ks1
    for i in range(5):
        for r in rot[i % 2]:
            x0 = x0 + x1
            x1 = (x1 << r) | (x1 >> (32 - r))
            x1 = x0 ^ x1
        x0 = x0 + ks[(i + 1) % 3]
        x1 = x1 + ks[(i + 2) % 3] + jnp.uint32(i + 1)
    return x0 ^ x1


def _final_body(gath_ref, baseg_ref, lowg_ref, thr_ref, o_ref, vw_ref, gg_ref):
    x = gath_ref[:, 0, :]
    thr = thr_ref[:, :1]
    off512 = lax.broadcasted_iota(jnp.int32, (B, C), 1)
    gcols = []
    lcols = []
    for r in range(NSEL):
        gcols.append(baseg_ref[:, r * 16:r * 16 + 1] + off512)
        lcols.append(jnp.broadcast_to(lowg_ref[:, r * 16:r * 16 + 1], (B, C)))
    gg = jnp.concatenate(gcols, axis=1)
    lo = jnp.concatenate(lcols, axis=1)
    ok = (x >= thr) & (gg >= lo) & (gg < V)
    vw_ref[...] = jnp.where(ok, x, NEG_INF) / TEMP
    gg_ref[...] = jnp.where(ok, gg, IMAX)

    lane = lax.broadcasted_iota(jnp.int32, (B, 64), 1)

    def wstep(i, carry):
        sv, sg = carry
        vw = vw_ref[...]
        ggm = gg_ref[...]
        m = jnp.max(vw, axis=1, keepdims=True)
        gsel = jnp.min(jnp.where(vw == m, ggm, IMAX), axis=1, keepdims=True)
        sv = jnp.where(lane == i, m, sv)
        sg = jnp.where(lane == i, gsel, sg)
        vw_ref[...] = jnp.where((vw == m) & (ggm == gsel), NEG_INF, vw)
        return sv, sg

    sv0 = jnp.minimum(x[:, :64], NEG_INF)
    sg0 = jnp.maximum(gg[:, :64], IMAX)
    vraw, g = lax.fori_loop(0, K, wstep, (sv0, sg0))
    v = vraw

    def stepA(i, carry):
        vw, kept = carry
        m = jnp.max(vw, axis=1, keepdims=True)
        gsel = jnp.min(jnp.where(vw == m, g, IMAX), axis=1, keepdims=True)
        hit = (vw == m) & (g == gsel)
        kept = jnp.where(hit, jnp.int32(1), kept)
        vw = jnp.where(hit, NEG_INF, vw)
        return vw, kept

    _, kept_i = lax.fori_loop(0, K, stepA, (v, g * 0))
    kept = kept_i == 1

    vk = jnp.where(kept, v, NEG_INF)

    def stepB(i, carry):
        vw, sv, sg = carry
        m = jnp.max(vw, axis=1, keepdims=True)
        gsel = jnp.max(jnp.where(vw == m, g, jnp.int32(-1)), axis=1,
                       keepdims=True)
        sv = jnp.where(lane == i, m, sv)
        sg = jnp.where(lane == i, gsel, sg)
        vw = jnp.where((vw == m) & (g == gsel), NEG_INF, vw)
        return vw, sv, sg

    _, sv, sg = lax.fori_loop(
        0, K, stepB, (vk, jnp.minimum(v, NEG_INF), jnp.maximum(g, IMAX)))
    valid = lane < K

    m0 = jnp.max(sv, axis=1, keepdims=True)
    e = jnp.where(valid, jnp.exp(sv - m0), np.float32(0.0))
    s = jnp.sum(e, axis=1, keepdims=True)
    p = e / s

    def cstep(j, carry):
        run, cprev = carry
        cprev = jnp.where(lane == j, run, cprev)
        pj = jnp.sum(jnp.where(lane == j, p, np.float32(0.0)), axis=1,
                     keepdims=True)
        return run + pj, cprev

    _, cprev = lax.fori_loop(
        0, K, cstep,
        (jnp.sum(p, axis=1, keepdims=True) * np.float32(0.0),
         p * np.float32(0.0)))
    remove = (lane > 0) & (cprev > TOP_P)
    keep = valid & jnp.logical_not(remove)

    row = lax.broadcasted_iota(jnp.int32, (B, 64), 0)
    pflat = (row * V + jnp.where(keep, sg, 0)).astype(jnp.uint32)
    bits = _threefry_bits(pflat)
    fb = (bits >> 9) | jnp.uint32(np.float32(1.0).view(np.uint32))
    f = lax.bitcast_convert_type(fb, jnp.float32) - np.float32(1.0)
    tiny = np.float32(np.finfo(np.float32).tiny)
    u = jnp.maximum(tiny, f * (np.float32(1.0) - tiny) + tiny)
    gum = -jnp.log(-jnp.log(u))
    score = jnp.where(keep, sv + gum, NEG_INF)
    ms = jnp.max(score, axis=1, keepdims=True)
    wg = jnp.min(jnp.where(score == ms, sg, IMAX), axis=1, keepdims=True)
    o_ref[...] = wg


def kernel(logits):
    cmax = pl.pallas_call(
        _cmax_body,
        grid=(NBLK,),
        in_specs=[pl.BlockSpec((B, 1, W), lambda i: (0, 0, i))],
        out_specs=pl.BlockSpec((B, W // C), lambda i: (0, i)),
        out_shape=jax.ShapeDtypeStruct((B, NCP), jnp.float32),
    )(logits)

    startg, baseg, lowg, thr = pl.pallas_call(
        _select_body,
        out_shape=[
            jax.ShapeDtypeStruct((B, 64), jnp.int32),
            jax.ShapeDtypeStruct((B, 1024), jnp.int32),
            jax.ShapeDtypeStruct((B, 1024), jnp.int32),
            jax.ShapeDtypeStruct((B, 16), jnp.float32),
        ],
    )(cmax)

    mesh = plsc.VectorSubcoreMesh(core_axis_name="c", subcore_axis_name="s")
    gathered = pl.kernel(
        _sc_body,
        mesh=mesh,
        out_type=jax.ShapeDtypeStruct((B, 1, NSEL * C), jnp.float32),
        scratch_types=[
            pltpu.VMEM((64,), jnp.int32),
            pltpu.VMEM((NSEL * C,), jnp.float32),
            pltpu.SemaphoreType.DMA,
        ],
    )(logits, startg)

    tok = pl.pallas_call(
        _final_body,
        out_shape=jax.ShapeDtypeStruct((B, 1), jnp.int32),
        scratch_shapes=[
            pltpu.VMEM((B, NSEL * C), jnp.float32),
            pltpu.VMEM((B, NSEL * C), jnp.int32),
        ],
    )(gathered, baseg, lowg, thr)
    return tok
